# baseline (speedup 1.0000x reference)
import functools

import jax
import jax.numpy as jnp
from jax import lax
from jax.experimental import pallas as pl
from jax.experimental.pallas import tpu as pltpu
from jax.experimental.pallas import tpu_sc as plsc

GRID = 64
NBINS = GRID ** 3
NBATCH = 4
NPTS = 500000
NTILES = 16
LANES = 16

P_TILE = NPTS // NTILES
W_TILE = 3 * P_TILE
CH_PTS = 6400
CH_W = 3 * CH_PTS
CH_GROUPS = CH_PTS // LANES
NFULL = 4
LAST_PTS = P_TILE - NFULL * CH_PTS
LAST_GROUPS = LAST_PTS // LANES
REM = LAST_PTS - LAST_GROUPS * LANES
LEN_CH = CH_W + 8
LEN_LAST = 16960
LEN_LAST15 = 16952
PBUF = 19264
ROW = 31264
IDXCH = (7808, 7824, 7808, 7824)
IDXOFF = (0, 7808, 15632, 23440)
IDXGRP = (488, 489, 488, 489)
QBINS = NBINS // 4
SHARE = NBINS // NTILES
SLICE = SHARE
PIECE = 4096
SENTINEL = 0x40000000
A_WORDS = QBINS
B_WORDS = 15648


def _body(pts, out, idxs, buf_f, buf_i, mmv, mma, hist_sh, mm_sh,
          sem_a, sem_b, sem_w):
    c = lax.axis_index("c")
    t = lax.axis_index("s")
    li = jnp.arange(LANES, dtype=jnp.int32)
    l3 = li * 3
    ones = jnp.full((LANES,), 1.0, dtype=jnp.float32)
    zero = jnp.zeros((LANES,), dtype=jnp.float32)
    inf = jnp.float32(jnp.inf)
    pinf = jnp.full((LANES,), inf, dtype=jnp.float32)
    ninf = jnp.full((LANES,), -inf, dtype=jnp.float32)

    w0 = t * W_TILE
    moff = jnp.bitwise_and(w0, 7)
    psems = [sem_a, sem_b]

    def gxyz(ref_base, g):
        base = ref_base + moff + g * (3 * LANES)
        xs = plsc.load_gather(buf_f, [base + l3])
        ys = plsc.load_gather(buf_f, [base + l3 + 1])
        zs = plsc.load_gather(buf_f, [base + l3 + 2])
        return xs, ys, zs

    for bo in range(2):
        b = c * 2 + bo
        bw = b * (NPTS * 3)

        def issue_pts(k):
            src = pl.multiple_of(bw + w0 + k * CH_W - moff, 8)
            dst = (k % 2) * PBUF
            return pltpu.async_copy(pts.at[pl.ds(src, LEN_CH)],
                                    buf_f.at[pl.ds(dst, LEN_CH)],
                                    psems[k % 2])

        def stage_last():
            src = pl.multiple_of(bw + w0 + NFULL * CH_W - moff, 8)
            dst = (NFULL % 2) * PBUF

            @pl.when(t < NTILES - 1)
            def _():
                pltpu.sync_copy(pts.at[pl.ds(src, LEN_LAST)],
                                buf_f.at[pl.ds(dst, LEN_LAST)])

            @pl.when(t == NTILES - 1)
            def _():
                pltpu.sync_copy(pts.at[pl.ds(src, LEN_LAST15)],
                                buf_f.at[pl.ds(dst, LEN_LAST15)])

        cr = (pinf, pinf, pinf, ninf, ninf, ninf)
        descs = [issue_pts(0)]
        for k in range(NFULL + 1):
            if k + 1 < NFULL:
                descs.append(issue_pts(k + 1))
            if k == NFULL:
                stage_last()
            else:
                descs[k].wait()
            rb = (k % 2) * PBUF
            ngrp = CH_GROUPS if k < NFULL else LAST_GROUPS

            def mm_body(g, cry, rb=rb):
                mnx, mny, mnz, mxx, mxy, mxz = cry
                xs, ys, zs = gxyz(rb, g)
                return (jnp.minimum(mnx, xs), jnp.minimum(mny, ys),
                        jnp.minimum(mnz, zs), jnp.maximum(mxx, xs),
                        jnp.maximum(mxy, ys), jnp.maximum(mxz, zs))

            cr = lax.fori_loop(0, ngrp, mm_body, cr, unroll=2)
        xs, ys, zs = gxyz((NFULL % 2) * PBUF, LAST_GROUPS)
        lmask = li < REM
        mnx = jnp.minimum(cr[0], jnp.where(lmask, xs, inf))
        mny = jnp.minimum(cr[1], jnp.where(lmask, ys, inf))
        mnz = jnp.minimum(cr[2], jnp.where(lmask, zs, inf))
        mxx = jnp.maximum(cr[3], jnp.where(lmask, xs, -inf))
        mxy = jnp.maximum(cr[4], jnp.where(lmask, ys, -inf))
        mxz = jnp.maximum(cr[5], jnp.where(lmask, zs, -inf))

        pub = jnp.where(li == 0, jnp.min(mnx),
              jnp.where(li == 1, jnp.min(mny),
              jnp.where(li == 2, jnp.min(mnz),
              jnp.where(li == 3, -jnp.max(mxx),
              jnp.where(li == 4, -jnp.max(mxy),
              jnp.where(li == 5, -jnp.max(mxz), inf))))))
        mmv[...] = pub
        pltpu.sync_copy(mmv, mm_sh.at[pl.ds(pl.multiple_of(t * LANES, 8),
                                            LANES)])
        plsc.subcore_barrier()
        pltpu.sync_copy(mm_sh, mma)
        acc = mma[pl.ds(0, LANES)]
        for j in range(1, NTILES):
            acc = jnp.minimum(acc, mma[pl.ds(j * LANES, LANES)])

        def lane_scalar(a):
            return jnp.min(jnp.where(li == a, acc, inf))

        pmin = [lane_scalar(0), lane_scalar(1), lane_scalar(2)]
        pmax = [-lane_scalar(3), -lane_scalar(4), -lane_scalar(5)]
        den = [(pmax[a] - pmin[a]) + jnp.float32(1e-6) for a in range(3)]

        def bin3(xs, ys, zs):
            def ax(p, a):
                tt = (p - pmin[a]) / den[a]
                n = tt * jnp.float32(2.0) - jnp.float32(1.0)
                u = (n + jnp.float32(1.0)) * jnp.float32(32.0)
                u = jnp.minimum(u, jnp.float32(63.0))
                return u.astype(jnp.int32)
            return ax(xs, 0) * 4096 + ax(ys, 1) * 64 + ax(zs, 2)

        rowbase = (c * NTILES + t) * ROW
        descs = [issue_pts(0)]
        wdescs = []
        for k in range(NFULL + 1):
            if k + 1 < NFULL:
                descs.append(issue_pts(k + 1))
            if k == NFULL:
                stage_last()
            else:
                descs[k].wait()
            if k >= 2:
                wdescs[k - 2].wait()
            rb = (k % 2) * PBUF
            ib = (k % 2) * CH_PTS
            ngrp = CH_GROUPS if k < NFULL else LAST_GROUPS

            def idx_body(g, cry, rb=rb, ib=ib):
                xs, ys, zs = gxyz(rb, g)
                buf_i[pl.ds(ib + g * LANES, LANES)] = bin3(xs, ys, zs)
                return cry

            lax.fori_loop(0, ngrp, idx_body, 0, unroll=2)
            if k < NFULL:
                nwords = CH_PTS
            else:
                xs, ys, zs = gxyz(rb, LAST_GROUPS)
                flat = bin3(xs, ys, zs)
                buf_i[pl.ds(ib + LAST_GROUPS * LANES, LANES)] = jnp.where(
                    li < REM, flat, jnp.int32(SENTINEL))
                nwords = (LAST_GROUPS + 1) * LANES
            wdescs.append(pltpu.async_copy(
                buf_i.at[pl.ds(ib, nwords)],
                idxs.at[pl.ds(pl.multiple_of(rowbase + k * CH_PTS, 8),
                              nwords)],
                sem_w))
        wdescs[NFULL - 1].wait()
        wdescs[NFULL].wait()
        plsc.subcore_barrier()

        q_lo = jnp.bitwise_and(t, 3) * QBINS
        g4 = t - jnp.bitwise_and(t, 3)

        def issue_idx(k):
            r, cc = k // 4, k % 4
            src = pl.multiple_of((c * NTILES + g4 + r) * ROW + IDXOFF[cc], 8)
            return pltpu.async_copy(
                idxs.at[pl.ds(src, IDXCH[cc])],
                buf_i.at[pl.ds((k % 2) * 7824, IDXCH[cc])],
                psems[k % 2])

        desc = [issue_idx(0)]

        def z_body(i, cry):
            buf_f[pl.ds(i * LANES, LANES)] = zero
            return cry

        lax.fori_loop(0, QBINS // LANES, z_body, 0, unroll=8)

        for k in range(16):
            if k + 1 < 16:
                desc.append(issue_idx(k + 1))
            desc[k].wait()
            cbase = (k % 2) * 7824

            def s_body(i, cry, cbase=cbase):
                v = buf_i[pl.ds(cbase + i * LANES, LANES)]
                rel = v - q_lo
                msk = plsc.bitcast(rel, jnp.uint32) < jnp.uint32(QBINS)
                plsc.addupdate_scatter(buf_f, [rel], ones, mask=msk)
                return cry

            lax.fori_loop(0, IDXGRP[k % 4], s_body, 0, unroll=4)

        qq = jnp.right_shift(t, 2)
        seg = jnp.bitwise_and(t, 3)
        for s in range(4):
            pltpu.sync_copy(
                buf_f.at[pl.ds(s * SLICE, SLICE)],
                hist_sh.at[pl.ds(pl.multiple_of(t * SLICE, 8), SLICE)])
            plsc.subcore_barrier()

            @pl.when(seg == s)
            def _(s=s):
                mbase = s * SLICE
                for p in range(SHARE // PIECE):
                    for j in range(4):
                        src = pl.multiple_of(
                            (4 * j + qq) * SLICE + p * PIECE, 8)
                        pltpu.sync_copy(
                            hist_sh.at[pl.ds(src, PIECE)],
                            buf_f.at[pl.ds(mbase + j * PIECE, PIECE)])

                    def m_body(i, cry, mbase=mbase):
                        o = mbase + i * LANES
                        ssum = (buf_f[pl.ds(o, LANES)]
                                + buf_f[pl.ds(PIECE + o, LANES)]
                                + buf_f[pl.ds(2 * PIECE + o, LANES)]
                                + buf_f[pl.ds(3 * PIECE + o, LANES)])
                        buf_f[pl.ds(o, LANES)] = ssum
                        return cry

                    lax.fori_loop(0, PIECE // LANES, m_body, 0, unroll=4)
                    dst = pl.multiple_of(
                        b * NBINS + t * SHARE + p * PIECE, 8)
                    pltpu.sync_copy(buf_f.at[pl.ds(mbase, PIECE)],
                                    out.at[pl.ds(dst, PIECE)])

            plsc.subcore_barrier()


_voxelize = functools.partial(
    pl.kernel,
    out_type=[
        jax.ShapeDtypeStruct((NBATCH * NBINS,), jnp.float32),
        jax.ShapeDtypeStruct((2 * NTILES * ROW,), jnp.int32),
    ],
    mesh=plsc.VectorSubcoreMesh(core_axis_name="c", subcore_axis_name="s"),
    compiler_params=pltpu.CompilerParams(needs_layout_passes=False),
    scratch_types=[
        pltpu.VMEM((A_WORDS,), jnp.float32),
        pltpu.VMEM((B_WORDS,), jnp.int32),
        pltpu.VMEM((LANES,), jnp.float32),
        pltpu.VMEM((NTILES * LANES,), jnp.float32),
        pltpu.VMEM_SHARED((NTILES * SLICE,), jnp.float32),
        pltpu.VMEM_SHARED((NTILES * LANES,), jnp.float32),
        pltpu.SemaphoreType.DMA,
        pltpu.SemaphoreType.DMA,
        pltpu.SemaphoreType.DMA,
    ],
)(_body)


def kernel(points):
    pts2 = points.reshape(NBATCH * NPTS * 3)
    out, _ = _voxelize(pts2)
    return out.reshape(NBATCH, 1, GRID, GRID, GRID)
